# baseline (speedup 1.0000x reference)
import functools

import jax
import jax.numpy as jnp
import numpy as np
from jax import lax
from jax.experimental import pallas as pl
from jax.experimental.pallas import tpu as pltpu
from jax.experimental.pallas import tpu_sc as plsc

_HARD_P = 0.5
_RAND_P = 0.1
_NSUB = 16
_NCORE = 2
_NBKT = 256
_LEVELS = ((30, 23), (22, 15), (14, 7), (6, 0))

_RAND_CACHE = {}


def _tf2x32(k1, k2, x0, x1):
    k1 = np.uint32(k1)
    k2 = np.uint32(k2)
    x0 = x0.astype(np.uint32).copy()
    x1 = x1.astype(np.uint32).copy()
    ks = [k1, k2, np.uint32(k1 ^ k2 ^ np.uint32(0x1BD11BDA))]
    rot = [np.uint32([13, 15, 26, 6]), np.uint32([17, 29, 16, 24])]

    def rotl(v, d):
        return (v << np.uint32(d)) | (v >> np.uint32(32 - d))

    with np.errstate(over="ignore"):
        x0 += ks[0]
        x1 += ks[1]
        inj = [(1, 2), (2, 0), (0, 1), (1, 2), (2, 0)]
        for g in range(5):
            for r in rot[g % 2]:
                x0 += x1
                x1 = rotl(x1, r)
                x1 ^= x0
            a, bb = inj[g]
            x0 += ks[a]
            x1 += ks[bb] + np.uint32(g + 1)
    return x0, x1


def _np_split(key, num):
    b1, b2 = _tf2x32(key[0], key[1], np.zeros(num, np.uint32),
                     np.arange(num, dtype=np.uint32))
    return [(b1[i], b2[i]) for i in range(num)]


def _np_shuffle(key, x):
    n = x.size
    num_rounds = int(np.ceil(3 * np.log(max(1, n)) / np.log(2**32 - 1)))
    for _ in range(num_rounds):
        key, sub = _np_split(key, 2)
        b1, b2 = _tf2x32(sub[0], sub[1], np.zeros(n, np.uint32),
                         np.arange(n, dtype=np.uint32))
        x = x[np.argsort(b1 ^ b2, kind="stable")]
    return x


def _random_mask_const(b, hw):
    key = (b, hw)
    if key not in _RAND_CACHE:
        n_ones = int(_RAND_P * hw)
        base = np.concatenate([np.ones(n_ones, np.float32),
                               np.zeros(hw - n_ones, np.float32)])
        keys = _np_split((np.uint32(0), np.uint32(42)), b)
        _RAND_CACHE[key] = np.stack([_np_shuffle(k, base) for k in keys])
    return _RAND_CACHE[key]


def _res_body(x_ref, y_ref, rand_ref, mag_ref, sgn_ref):
    s = jnp.sum(jnp.abs(x_ref[...] - y_ref[...]), axis=1)
    mag_ref[...] = jax.lax.bitcast_convert_type(s, jnp.int32)
    sgn_ref[...] = jnp.where(rand_ref[...] > 0.0, -s, s)


def _residual_signed(x, y, rand):
    b, c, h, w = x.shape
    rows = 64
    grid = (b, h // rows)
    return pl.pallas_call(
        _res_body,
        grid=grid,
        in_specs=[
            pl.BlockSpec((1, c, rows, w), lambda i, j: (i, 0, j, 0)),
            pl.BlockSpec((1, c, rows, w), lambda i, j: (i, 0, j, 0)),
            pl.BlockSpec((1, rows, w), lambda i, j: (i, j, 0)),
        ],
        out_specs=[
            pl.BlockSpec((1, rows, w), lambda i, j: (i, j, 0)),
            pl.BlockSpec((1, rows, w), lambda i, j: (i, j, 0)),
        ],
        out_shape=[
            jax.ShapeDtypeStruct((b, h, w), jnp.int32),
            jax.ShapeDtypeStruct((b, h, w), jnp.float32),
        ],
    )(x, y, rand)


def _sc_select_sum(resmag, ressgn, kth, inv_n):
    b, hw = resmag.shape
    bpc = b // _NCORE
    chunk = hw // _NSUB
    nvec = chunk // 16
    mesh = plsc.VectorSubcoreMesh(core_axis_name="c", subcore_axis_name="s",
                                  num_cores=_NCORE, num_subcores=_NSUB)

    @functools.partial(
        pl.kernel,
        out_type=jax.ShapeDtypeStruct((_NCORE, 16), jnp.float32),
        mesh=mesh,
        compiler_params=pltpu.CompilerParams(needs_layout_passes=False),
        scratch_types=[
            pltpu.VMEM((chunk,), jnp.int32),
            pltpu.VMEM((chunk,), jnp.float32),
            pltpu.VMEM((16 * _NBKT,), jnp.int32),
            pltpu.VMEM((_NBKT,), jnp.int32),
            pltpu.VMEM((_NSUB, _NBKT), jnp.int32),
            pltpu.VMEM((_NSUB, 256), jnp.float32),
            pltpu.VMEM((256,), jnp.float32),
            pltpu.VMEM((16,), jnp.float32),
            pltpu.VMEM_SHARED((bpc * len(_LEVELS), _NSUB, _NBKT), jnp.int32),
            pltpu.VMEM_SHARED((_NSUB, 256), jnp.float32),
        ],
    )
    def k(resmag_hbm, ressgn_hbm, out_hbm, ints_v, res_v, lhist_v, chist_v,
          allhist_v, allacc_v, stage_v, ostage_v, hist_sh, acc_sh):
        cid = lax.axis_index("c")
        sid = lax.axis_index("s")
        lanes = lax.iota(jnp.int32, 16)
        ones = jnp.ones((16,), jnp.int32)
        acc = jnp.zeros((16,), jnp.float32)
        for t in range(bpc):
            bidx = cid * bpc + t
            pltpu.sync_copy(resmag_hbm.at[bidx, pl.ds(sid * chunk, chunk)],
                            ints_v)
            pltpu.sync_copy(ressgn_hbm.at[bidx, pl.ds(sid * chunk, chunk)],
                            res_v)
            prefix = jnp.int32(0)
            krem = jnp.int32(kth)
            for lvl, (hi, lo) in enumerate(_LEVELS):
                nbits = hi - lo + 1

                def zero_body(i, _):
                    lhist_v[pl.ds(i * 16, 16)] = jnp.zeros((16,), jnp.int32)
                    return 0

                lax.fori_loop(0, _NBKT, zero_body, 0, unroll=8)

                def hist_body(i, _, hi=hi, lo=lo, nbits=nbits, prefix=prefix):
                    v = ints_v[pl.ds(i * 16, 16)]
                    match = (v >> (hi + 1)) == (prefix >> (hi + 1))
                    field = (v >> lo) & jnp.int32((1 << nbits) - 1)
                    plsc.addupdate_scatter(
                        lhist_v, [lanes * _NBKT + field], ones, mask=match)
                    return 0

                lax.fori_loop(0, nvec, hist_body, 0, unroll=8)

                def fold_body(j, _):
                    s = jnp.zeros((16,), jnp.int32)
                    for l in range(16):
                        s = s + lhist_v[pl.ds(l * _NBKT + j * 16, 16)]
                    chist_v[pl.ds(j * 16, 16)] = s
                    return 0

                lax.fori_loop(0, _NBKT // 16, fold_body, 0, unroll=4)

                slot = t * len(_LEVELS) + lvl
                pltpu.sync_copy(chist_v, hist_sh.at[slot, sid])
                plsc.subcore_barrier()
                pltpu.sync_copy(hist_sh.at[slot], allhist_v)

                def scan_body(i, carry, krem=krem):
                    cnt, found, fstar, s_above = carry
                    j = _NBKT // 16 - 1 - i
                    h = jnp.zeros((16,), jnp.int32)
                    for r in range(_NSUB):
                        h = h + allhist_v[r, pl.ds(j * 16, 16)]
                    hrev = jnp.flip(h)
                    cw = jnp.cumsum(hrev) + cnt
                    hit = cw >= krem + 1
                    nhit = jnp.sum(hit.astype(jnp.int32))
                    fl = 16 - nhit
                    onehot = lanes == fl
                    cumsel = jnp.sum(jnp.where(onehot, cw, 0))
                    hsel = jnp.sum(jnp.where(onehot, hrev, 0))
                    newly = jnp.logical_and(found == 0, nhit > 0)
                    fstar = jnp.where(newly, j * 16 + 15 - fl, fstar)
                    s_above = jnp.where(newly, cumsel - hsel, s_above)
                    found = jnp.where(nhit > 0, jnp.int32(1), found)
                    cnt = cnt + jnp.sum(h)
                    return cnt, found, fstar, s_above

                z = jnp.int32(0)
                _, _, fstar, s_above = lax.fori_loop(
                    0, _NBKT // 16, scan_body, (z, z, z, z))
                prefix = prefix | (fstar << lo)
                krem = krem - s_above

            thre = prefix

            def sum_body(i, a, thre=thre):
                v = ints_v[pl.ds(i * 16, 16)]
                rf = res_v[pl.ds(i * 16, 16)]
                keep = (v > thre) | (rf < 0.0)
                return a + jnp.where(keep, jnp.abs(rf), jnp.float32(0.0))

            acc = lax.fori_loop(0, nvec, sum_body, acc, unroll=8)

        stage_v[pl.ds(0, 16)] = acc
        pltpu.sync_copy(stage_v, acc_sh.at[sid])
        plsc.subcore_barrier()

        @pl.when(sid == 0)
        def _():
            pltpu.sync_copy(acc_sh, allacc_v)
            tot = jnp.zeros((16,), jnp.float32)
            for r in range(_NSUB):
                tot = tot + allacc_v[r, pl.ds(0, 16)]
            total = jnp.sum(tot) * inv_n
            ostage_v[...] = jnp.where(lanes == 0, total, jnp.float32(0.0))
            pltpu.sync_copy(ostage_v, out_hbm.at[cid])

    return k(resmag, ressgn)


def kernel(x, y):
    b, c, h, w = x.shape
    hw = h * w
    kth = int(_HARD_P * hw)
    rand = jnp.asarray(_random_mask_const(b, hw).reshape(b, h, w))
    resmag, ressgn = _residual_signed(x, y, rand)
    out = _sc_select_sum(resmag.reshape(b, hw), ressgn.reshape(b, hw), kth,
                         np.float32(1.0 / (b * c * hw)))
    return out[0, 0] + out[1, 0]


# speedup vs baseline: 1.0043x; 1.0043x over previous
import functools

import jax
import jax.numpy as jnp
import numpy as np
from jax import lax
from jax.experimental import pallas as pl
from jax.experimental.pallas import tpu as pltpu
from jax.experimental.pallas import tpu_sc as plsc

_HARD_P = 0.5
_RAND_P = 0.1
_NSUB = 16
_NCORE = 2
_NBKT = 256
_LEVELS = ((30, 23), (22, 15), (14, 7), (6, 0))

_RAND_CACHE = {}


def _tf2x32(k1, k2, x0, x1):
    k1 = np.uint32(k1)
    k2 = np.uint32(k2)
    x0 = x0.astype(np.uint32).copy()
    x1 = x1.astype(np.uint32).copy()
    ks = [k1, k2, np.uint32(k1 ^ k2 ^ np.uint32(0x1BD11BDA))]
    rot = [np.uint32([13, 15, 26, 6]), np.uint32([17, 29, 16, 24])]

    def rotl(v, d):
        return (v << np.uint32(d)) | (v >> np.uint32(32 - d))

    with np.errstate(over="ignore"):
        x0 += ks[0]
        x1 += ks[1]
        inj = [(1, 2), (2, 0), (0, 1), (1, 2), (2, 0)]
        for g in range(5):
            for r in rot[g % 2]:
                x0 += x1
                x1 = rotl(x1, r)
                x1 ^= x0
            a, bb = inj[g]
            x0 += ks[a]
            x1 += ks[bb] + np.uint32(g + 1)
    return x0, x1


def _np_split(key, num):
    b1, b2 = _tf2x32(key[0], key[1], np.zeros(num, np.uint32),
                     np.arange(num, dtype=np.uint32))
    return [(b1[i], b2[i]) for i in range(num)]


def _np_shuffle(key, x):
    n = x.size
    num_rounds = int(np.ceil(3 * np.log(max(1, n)) / np.log(2**32 - 1)))
    for _ in range(num_rounds):
        key, sub = _np_split(key, 2)
        b1, b2 = _tf2x32(sub[0], sub[1], np.zeros(n, np.uint32),
                         np.arange(n, dtype=np.uint32))
        x = x[np.argsort(b1 ^ b2, kind="stable")]
    return x


def _random_mask_const(b, hw):
    key = (b, hw)
    if key not in _RAND_CACHE:
        n_ones = int(_RAND_P * hw)
        base = np.concatenate([np.ones(n_ones, np.float32),
                               np.zeros(hw - n_ones, np.float32)])
        keys = _np_split((np.uint32(0), np.uint32(42)), b)
        _RAND_CACHE[key] = np.stack([_np_shuffle(k, base) for k in keys])
    return _RAND_CACHE[key]


def _res_body(x_ref, y_ref, rand_ref, mag_ref, sgn_ref):
    s = jnp.sum(jnp.abs(x_ref[...] - y_ref[...]), axis=1)
    mag_ref[...] = jax.lax.bitcast_convert_type(s, jnp.int32)
    sgn_ref[...] = jnp.where(rand_ref[...] > 0.0, -s, s)


def _residual_signed(x, y, rand):
    b, c, h, w = x.shape
    rows = 16
    grid = (b, h // rows)
    return pl.pallas_call(
        _res_body,
        grid=grid,
        in_specs=[
            pl.BlockSpec((1, c, rows, w), lambda i, j: (i, 0, j, 0)),
            pl.BlockSpec((1, c, rows, w), lambda i, j: (i, 0, j, 0)),
            pl.BlockSpec((1, rows, w), lambda i, j: (i, j, 0)),
        ],
        out_specs=[
            pl.BlockSpec((1, rows, w), lambda i, j: (i, j, 0)),
            pl.BlockSpec((1, rows, w), lambda i, j: (i, j, 0)),
        ],
        out_shape=[
            jax.ShapeDtypeStruct((b, h, w), jnp.int32),
            jax.ShapeDtypeStruct((b, h, w), jnp.float32),
        ],
    )(x, y, rand)


def _sc_select_sum(resmag, ressgn, kth, inv_n):
    b, hw = resmag.shape
    bpc = b // _NCORE
    chunk = hw // _NSUB
    nvec = chunk // 16
    mesh = plsc.VectorSubcoreMesh(core_axis_name="c", subcore_axis_name="s",
                                  num_cores=_NCORE, num_subcores=_NSUB)

    @functools.partial(
        pl.kernel,
        out_type=jax.ShapeDtypeStruct((_NCORE, 16), jnp.float32),
        mesh=mesh,
        compiler_params=pltpu.CompilerParams(needs_layout_passes=False),
        scratch_types=[
            pltpu.VMEM((chunk,), jnp.int32),
            pltpu.VMEM((chunk,), jnp.float32),
            pltpu.VMEM((16 * _NBKT,), jnp.int32),
            pltpu.VMEM((_NBKT,), jnp.int32),
            pltpu.VMEM((_NSUB, _NBKT), jnp.int32),
            pltpu.VMEM((_NSUB, 256), jnp.float32),
            pltpu.VMEM((256,), jnp.float32),
            pltpu.VMEM((16,), jnp.float32),
            pltpu.VMEM_SHARED((bpc * len(_LEVELS), _NSUB, _NBKT), jnp.int32),
            pltpu.VMEM_SHARED((_NSUB, 256), jnp.float32),
        ],
    )
    def k(resmag_hbm, ressgn_hbm, out_hbm, ints_v, res_v, lhist_v, chist_v,
          allhist_v, allacc_v, stage_v, ostage_v, hist_sh, acc_sh):
        cid = lax.axis_index("c")
        sid = lax.axis_index("s")
        lanes = lax.iota(jnp.int32, 16)
        ones = jnp.ones((16,), jnp.int32)
        acc = jnp.zeros((16,), jnp.float32)
        for t in range(bpc):
            bidx = cid * bpc + t
            pltpu.sync_copy(resmag_hbm.at[bidx, pl.ds(sid * chunk, chunk)],
                            ints_v)
            pltpu.sync_copy(ressgn_hbm.at[bidx, pl.ds(sid * chunk, chunk)],
                            res_v)
            prefix = jnp.int32(0)
            krem = jnp.int32(kth)
            for lvl, (hi, lo) in enumerate(_LEVELS):
                nbits = hi - lo + 1

                def zero_body(i, _):
                    lhist_v[pl.ds(i * 16, 16)] = jnp.zeros((16,), jnp.int32)
                    return 0

                lax.fori_loop(0, _NBKT, zero_body, 0, unroll=8)

                def hist_body(i, _, hi=hi, lo=lo, nbits=nbits, prefix=prefix):
                    v = ints_v[pl.ds(i * 16, 16)]
                    match = (v >> (hi + 1)) == (prefix >> (hi + 1))
                    field = (v >> lo) & jnp.int32((1 << nbits) - 1)
                    plsc.addupdate_scatter(
                        lhist_v, [lanes * _NBKT + field], ones, mask=match)
                    return 0

                lax.fori_loop(0, nvec, hist_body, 0, unroll=8)

                def fold_body(j, _):
                    s = jnp.zeros((16,), jnp.int32)
                    for l in range(16):
                        s = s + lhist_v[pl.ds(l * _NBKT + j * 16, 16)]
                    chist_v[pl.ds(j * 16, 16)] = s
                    return 0

                lax.fori_loop(0, _NBKT // 16, fold_body, 0, unroll=4)

                slot = t * len(_LEVELS) + lvl
                pltpu.sync_copy(chist_v, hist_sh.at[slot, sid])
                plsc.subcore_barrier()
                pltpu.sync_copy(hist_sh.at[slot], allhist_v)

                def scan_body(i, carry, krem=krem):
                    cnt, found, fstar, s_above = carry
                    j = _NBKT // 16 - 1 - i
                    h = jnp.zeros((16,), jnp.int32)
                    for r in range(_NSUB):
                        h = h + allhist_v[r, pl.ds(j * 16, 16)]
                    hrev = jnp.flip(h)
                    cw = jnp.cumsum(hrev) + cnt
                    hit = cw >= krem + 1
                    nhit = jnp.sum(hit.astype(jnp.int32))
                    fl = 16 - nhit
                    onehot = lanes == fl
                    cumsel = jnp.sum(jnp.where(onehot, cw, 0))
                    hsel = jnp.sum(jnp.where(onehot, hrev, 0))
                    newly = jnp.logical_and(found == 0, nhit > 0)
                    fstar = jnp.where(newly, j * 16 + 15 - fl, fstar)
                    s_above = jnp.where(newly, cumsel - hsel, s_above)
                    found = jnp.where(nhit > 0, jnp.int32(1), found)
                    cnt = cnt + jnp.sum(h)
                    return cnt, found, fstar, s_above

                z = jnp.int32(0)
                _, _, fstar, s_above = lax.fori_loop(
                    0, _NBKT // 16, scan_body, (z, z, z, z))
                prefix = prefix | (fstar << lo)
                krem = krem - s_above

            thre = prefix

            def sum_body(i, a, thre=thre):
                v = ints_v[pl.ds(i * 16, 16)]
                rf = res_v[pl.ds(i * 16, 16)]
                keep = (v > thre) | (rf < 0.0)
                return a + jnp.where(keep, jnp.abs(rf), jnp.float32(0.0))

            acc = lax.fori_loop(0, nvec, sum_body, acc, unroll=8)

        stage_v[pl.ds(0, 16)] = acc
        pltpu.sync_copy(stage_v, acc_sh.at[sid])
        plsc.subcore_barrier()

        @pl.when(sid == 0)
        def _():
            pltpu.sync_copy(acc_sh, allacc_v)
            tot = jnp.zeros((16,), jnp.float32)
            for r in range(_NSUB):
                tot = tot + allacc_v[r, pl.ds(0, 16)]
            total = jnp.sum(tot) * inv_n
            ostage_v[...] = jnp.where(lanes == 0, total, jnp.float32(0.0))
            pltpu.sync_copy(ostage_v, out_hbm.at[cid])

    return k(resmag, ressgn)


def kernel(x, y):
    b, c, h, w = x.shape
    hw = h * w
    kth = int(_HARD_P * hw)
    rand = jnp.asarray(_random_mask_const(b, hw).reshape(b, h, w))
    resmag, ressgn = _residual_signed(x, y, rand)
    out = _sc_select_sum(resmag.reshape(b, hw), ressgn.reshape(b, hw), kth,
                         np.float32(1.0 / (b * c * hw)))
    return out[0, 0] + out[1, 0]


# speedup vs baseline: 1.0651x; 1.0605x over previous
import functools

import jax
import jax.numpy as jnp
import numpy as np
from jax import lax
from jax.experimental import pallas as pl
from jax.experimental.pallas import tpu as pltpu
from jax.experimental.pallas import tpu_sc as plsc

_HARD_P = 0.5
_RAND_P = 0.1
_NSUB = 16
_NCORE = 2
_NBKT = 256
_LEVELS = ((30, 23), (22, 15), (14, 7), (6, 0))

_RAND_CACHE = {}


def _tf2x32(k1, k2, x0, x1):
    k1 = np.uint32(k1)
    k2 = np.uint32(k2)
    x0 = x0.astype(np.uint32).copy()
    x1 = x1.astype(np.uint32).copy()
    ks = [k1, k2, np.uint32(k1 ^ k2 ^ np.uint32(0x1BD11BDA))]
    rot = [np.uint32([13, 15, 26, 6]), np.uint32([17, 29, 16, 24])]

    def rotl(v, d):
        return (v << np.uint32(d)) | (v >> np.uint32(32 - d))

    with np.errstate(over="ignore"):
        x0 += ks[0]
        x1 += ks[1]
        inj = [(1, 2), (2, 0), (0, 1), (1, 2), (2, 0)]
        for g in range(5):
            for r in rot[g % 2]:
                x0 += x1
                x1 = rotl(x1, r)
                x1 ^= x0
            a, bb = inj[g]
            x0 += ks[a]
            x1 += ks[bb] + np.uint32(g + 1)
    return x0, x1


def _np_split(key, num):
    b1, b2 = _tf2x32(key[0], key[1], np.zeros(num, np.uint32),
                     np.arange(num, dtype=np.uint32))
    return [(b1[i], b2[i]) for i in range(num)]


def _np_shuffle(key, x):
    n = x.size
    num_rounds = int(np.ceil(3 * np.log(max(1, n)) / np.log(2**32 - 1)))
    for _ in range(num_rounds):
        key, sub = _np_split(key, 2)
        b1, b2 = _tf2x32(sub[0], sub[1], np.zeros(n, np.uint32),
                         np.arange(n, dtype=np.uint32))
        x = x[np.argsort(b1 ^ b2, kind="stable")]
    return x


def _random_mask_const(b, hw):
    key = (b, hw)
    if key not in _RAND_CACHE:
        n_ones = int(_RAND_P * hw)
        base = np.concatenate([np.ones(n_ones, np.float32),
                               np.zeros(hw - n_ones, np.float32)])
        keys = _np_split((np.uint32(0), np.uint32(42)), b)
        _RAND_CACHE[key] = np.stack([_np_shuffle(k, base) for k in keys])
    return _RAND_CACHE[key]


def _res_body(x_ref, y_ref, rand_ref, mag_ref, sgn_ref):
    s = jnp.sum(jnp.abs(x_ref[...] - y_ref[...]), axis=1)
    mag_ref[...] = jax.lax.bitcast_convert_type(s, jnp.int32)
    sgn_ref[...] = jnp.where(rand_ref[...] > 0.0, -s, s)


def _residual_signed(x, y, rand):
    b, c, h, w = x.shape
    rows = 32
    grid = (b, h // rows)
    return pl.pallas_call(
        _res_body,
        grid=grid,
        in_specs=[
            pl.BlockSpec((1, c, rows, w), lambda i, j: (i, 0, j, 0)),
            pl.BlockSpec((1, c, rows, w), lambda i, j: (i, 0, j, 0)),
            pl.BlockSpec((1, rows, w), lambda i, j: (i, j, 0)),
        ],
        out_specs=[
            pl.BlockSpec((1, rows, w), lambda i, j: (i, j, 0)),
            pl.BlockSpec((1, rows, w), lambda i, j: (i, j, 0)),
        ],
        out_shape=[
            jax.ShapeDtypeStruct((b, h, w), jnp.int32),
            jax.ShapeDtypeStruct((b, h, w), jnp.float32),
        ],
    )(x, y, rand)


def _sc_select_sum(resmag, ressgn, kth, inv_n):
    b, hw = resmag.shape
    bpc = b // _NCORE
    chunk = hw // _NSUB
    nvec = chunk // 16
    mesh = plsc.VectorSubcoreMesh(core_axis_name="c", subcore_axis_name="s",
                                  num_cores=_NCORE, num_subcores=_NSUB)

    assert bpc == 2
    scratch_types = [
            pltpu.VMEM((2, chunk), jnp.int32),
            pltpu.VMEM((2, chunk), jnp.float32),
            pltpu.VMEM((16 * _NBKT,), jnp.int32),
            pltpu.VMEM((_NBKT,), jnp.int32),
            pltpu.VMEM((_NSUB, _NBKT), jnp.int32),
            pltpu.VMEM((_NSUB, 256), jnp.float32),
            pltpu.VMEM((256,), jnp.float32),
            pltpu.VMEM((16,), jnp.float32),
            pltpu.SemaphoreType.DMA,
            pltpu.VMEM_SHARED((bpc * len(_LEVELS), _NSUB, _NBKT), jnp.int32),
            pltpu.VMEM_SHARED((_NSUB, 256), jnp.float32),
    ]

    @functools.partial(
        pl.kernel,
        out_type=jax.ShapeDtypeStruct((_NCORE, 16), jnp.float32),
        mesh=mesh,
        compiler_params=pltpu.CompilerParams(needs_layout_passes=False),
        scratch_types=scratch_types,
    )
    def k(resmag_hbm, ressgn_hbm, out_hbm, ints_v, res_v, lhist_v, chist_v,
          allhist_v, allacc_v, stage_v, ostage_v, sem, hist_sh, acc_sh):
        cid = lax.axis_index("c")
        sid = lax.axis_index("s")
        lanes = lax.iota(jnp.int32, 16)
        ones = jnp.ones((16,), jnp.int32)

        copies = []
        for t in range(bpc):
            bidx = cid * bpc + t
            copies.append(pltpu.async_copy(
                resmag_hbm.at[bidx, pl.ds(sid * chunk, chunk)],
                ints_v.at[t], sem))
            copies.append(pltpu.async_copy(
                ressgn_hbm.at[bidx, pl.ds(sid * chunk, chunk)],
                res_v.at[t], sem))
        for cp in copies:
            cp.wait()

        prefix = [jnp.int32(0)] * bpc
        krem = [jnp.int32(kth)] * bpc
        for lvl, (hi, lo) in enumerate(_LEVELS):
            nbits = hi - lo + 1
            for t in range(bpc):

                def zero_body(i, _):
                    lhist_v[pl.ds(i * 16, 16)] = jnp.zeros((16,), jnp.int32)
                    return 0

                lax.fori_loop(0, _NBKT, zero_body, 0, unroll=8)

                def hist_body(i, _, hi=hi, lo=lo, nbits=nbits,
                              prefix=prefix[t], t=t):
                    v = ints_v[t, pl.ds(i * 16, 16)]
                    match = (v >> (hi + 1)) == (prefix >> (hi + 1))
                    field = (v >> lo) & jnp.int32((1 << nbits) - 1)
                    plsc.addupdate_scatter(
                        lhist_v, [lanes * _NBKT + field], ones, mask=match)
                    return 0

                lax.fori_loop(0, nvec, hist_body, 0, unroll=8)

                def fold_body(j, _):
                    s = jnp.zeros((16,), jnp.int32)
                    for l in range(16):
                        s = s + lhist_v[pl.ds(l * _NBKT + j * 16, 16)]
                    chist_v[pl.ds(j * 16, 16)] = s
                    return 0

                lax.fori_loop(0, _NBKT // 16, fold_body, 0, unroll=4)

                slot = t * len(_LEVELS) + lvl
                pltpu.sync_copy(chist_v, hist_sh.at[slot, sid])

            plsc.subcore_barrier()

            for t in range(bpc):
                slot = t * len(_LEVELS) + lvl
                pltpu.sync_copy(hist_sh.at[slot], allhist_v)

                def scan_body(i, carry, krem=krem[t]):
                    cnt, found, fstar, s_above = carry
                    j = _NBKT // 16 - 1 - i
                    h = jnp.zeros((16,), jnp.int32)
                    for r in range(_NSUB):
                        h = h + allhist_v[r, pl.ds(j * 16, 16)]
                    hrev = jnp.flip(h)
                    cw = jnp.cumsum(hrev) + cnt
                    hit = cw >= krem + 1
                    nhit = jnp.sum(hit.astype(jnp.int32))
                    fl = 16 - nhit
                    onehot = lanes == fl
                    cumsel = jnp.sum(jnp.where(onehot, cw, 0))
                    hsel = jnp.sum(jnp.where(onehot, hrev, 0))
                    newly = jnp.logical_and(found == 0, nhit > 0)
                    fstar = jnp.where(newly, j * 16 + 15 - fl, fstar)
                    s_above = jnp.where(newly, cumsel - hsel, s_above)
                    found = jnp.where(nhit > 0, jnp.int32(1), found)
                    cnt = cnt + jnp.sum(h)
                    return cnt, found, fstar, s_above

                z = jnp.int32(0)
                _, _, fstar, s_above = lax.fori_loop(
                    0, _NBKT // 16, scan_body, (z, z, z, z))
                prefix[t] = prefix[t] | (fstar << lo)
                krem[t] = krem[t] - s_above

        acc = jnp.zeros((16,), jnp.float32)
        for t in range(bpc):

            def sum_body(i, a, thre=prefix[t], t=t):
                v = ints_v[t, pl.ds(i * 16, 16)]
                rf = res_v[t, pl.ds(i * 16, 16)]
                keep = (v > thre) | (rf < 0.0)
                return a + jnp.where(keep, jnp.abs(rf), jnp.float32(0.0))

            acc = lax.fori_loop(0, nvec, sum_body, acc, unroll=8)

        stage_v[pl.ds(0, 16)] = acc
        pltpu.sync_copy(stage_v, acc_sh.at[sid])
        plsc.subcore_barrier()

        @pl.when(sid == 0)
        def _():
            pltpu.sync_copy(acc_sh, allacc_v)
            tot = jnp.zeros((16,), jnp.float32)
            for r in range(_NSUB):
                tot = tot + allacc_v[r, pl.ds(0, 16)]
            total = jnp.sum(tot) * inv_n
            ostage_v[...] = jnp.where(lanes == 0, total, jnp.float32(0.0))
            pltpu.sync_copy(ostage_v, out_hbm.at[cid])

    return k(resmag, ressgn)


def kernel(x, y):
    b, c, h, w = x.shape
    hw = h * w
    kth = int(_HARD_P * hw)
    rand = jnp.asarray(_random_mask_const(b, hw).reshape(b, h, w))
    resmag, ressgn = _residual_signed(x, y, rand)
    out = _sc_select_sum(resmag.reshape(b, hw), ressgn.reshape(b, hw), kth,
                         np.float32(1.0 / (b * c * hw)))
    return out[0, 0] + out[1, 0]
